# baseline (speedup 1.0000x reference)
import functools

import jax
import jax.numpy as jnp
from jax.experimental import pallas as pl
from jax.experimental.pallas import tpu as pltpu


def _head_body(xp_ref, wa_ref, ba_ref, w1_ref, b1_ref, w2_ref, b2_ref,
               wp_ref, bp_ref, o_ref, h0, h1, *, H, W, W_o, W_im):
    C = wa_ref.shape[0]
    F = wa_ref.shape[1]
    P = wp_ref.shape[1]
    M = H * W_o

    col = jax.lax.broadcasted_iota(jnp.int32, (M, F), 0) % W_o
    keep = col < W

    x = xp_ref[0, 1:H + 1, 1:W_o + 1, :].reshape(M, C)
    a = jnp.dot(x, wa_ref[:, :], preferred_element_type=jnp.float32)
    a = a + ba_ref[:, :]
    a = jnp.where(keep, a, 0.0)
    h0[...] = jnp.zeros((H + 2, W_im, C), jnp.float32)
    h0[1:H + 1, 1:W_o + 1, :] = a.reshape(H, W_o, C)

    def conv3(src_ref, w_ref, b_ref):
        acc = None
        for t in range(9):
            di, dj = t // 3, t % 3
            xs = src_ref[di:di + H, dj:dj + W_o, :].reshape(M, C)
            d = jnp.dot(xs, w_ref[t], preferred_element_type=jnp.float32)
            acc = d if acc is None else acc + d
        return jnp.maximum(acc + b_ref[:, :], 0.0)

    v1 = jnp.where(keep, conv3(h0, w1_ref, b1_ref), 0.0)
    h1[...] = jnp.zeros((H + 2, W_im, C), jnp.float32)
    h1[1:H + 1, 1:W_o + 1, :] = v1.reshape(H, W_o, C)
    v2 = conv3(h1, w2_ref, b2_ref)

    p = jnp.dot(v2, wp_ref[:, :], preferred_element_type=jnp.float32)
    p = p + bp_ref[:, :]
    k = jax.lax.broadcasted_iota(jnp.int32, (M, P), 1) % 15
    p = jnp.where(k == 4, jax.nn.sigmoid(p),
                  jnp.where(k == 14, jax.nn.softplus(p) + 1.0, p))
    o_ref[0] = p.reshape(H, W_o, P)


def _head_level(xp, wa, ba, w1, b1, w2, b2, wp, bp, H, W, W_o, W_im):
    B = xp.shape[0]
    C = xp.shape[-1]
    F = wa.shape[1]
    P = wp.shape[1]
    body = functools.partial(_head_body, H=H, W=W, W_o=W_o, W_im=W_im)
    return pl.pallas_call(
        body,
        grid=(B,),
        in_specs=[
            pl.BlockSpec((1, H + 2, W_im, C), lambda b: (b, 0, 0, 0)),
            pl.BlockSpec((C, F), lambda b: (0, 0)),
            pl.BlockSpec((1, F), lambda b: (0, 0)),
            pl.BlockSpec((9, C, F), lambda b: (0, 0, 0)),
            pl.BlockSpec((1, F), lambda b: (0, 0)),
            pl.BlockSpec((9, F, F), lambda b: (0, 0, 0)),
            pl.BlockSpec((1, F), lambda b: (0, 0)),
            pl.BlockSpec((F, P), lambda b: (0, 0)),
            pl.BlockSpec((1, P), lambda b: (0, 0)),
        ],
        out_specs=pl.BlockSpec((1, H, W_o, P), lambda b: (b, 0, 0, 0)),
        out_shape=jax.ShapeDtypeStruct((B, H, W_o, P), jnp.float32),
        scratch_shapes=[
            pltpu.VMEM((H + 2, W_im, C), jnp.float32),
            pltpu.VMEM((H + 2, W_im, C), jnp.float32),
        ],
        compiler_params=pltpu.CompilerParams(
            dimension_semantics=("arbitrary",)),
    )(xp, wa, ba, w1, b1, w2, b2, wp, bp)


def kernel(feat0, feat1, feat2, feat3, adapt_w, adapt_b, c1_w, c1_b, bn1_g,
           bn1_b, bn1_m, bn1_v, c2_w, c2_b, bn2_g, bn2_b, bn2_m, bn2_v,
           pred_w, pred_b):
    eps = 1e-5
    F = adapt_b.shape[1]
    C = feat0.shape[1]
    P = pred_b.shape[1]
    feats = [feat0, feat1, feat2, feat3]

    s1 = bn1_g / jnp.sqrt(bn1_v + eps)
    s2 = bn2_g / jnp.sqrt(bn2_v + eps)
    w1f = c1_w * s1[:, :, None, None, None]
    b1f = (c1_b - bn1_m) * s1 + bn1_b
    w2f = c2_w * s2[:, :, None, None, None]
    b2f = (c2_b - bn2_m) * s2 + bn2_b

    outs = []
    for i, f in enumerate(feats):
        B, _, H, W = f.shape
        W_o = -(-W // 8) * 8
        W_im = -(-(W_o + 2) // 8) * 8
        x = f.transpose(0, 2, 3, 1)
        xp = jnp.pad(x, ((0, 0), (1, 1), (1, W_im - W - 1), (0, 0)))
        wa = adapt_w[i].reshape(F, C).transpose(1, 0)
        ba = adapt_b[i].reshape(1, F)
        w1 = w1f[i].transpose(2, 3, 1, 0).reshape(9, C, F)
        b1 = b1f[i].reshape(1, F)
        w2 = w2f[i].transpose(2, 3, 1, 0).reshape(9, F, F)
        b2 = b2f[i].reshape(1, F)
        wp = pred_w[i].reshape(P, F).transpose(1, 0)
        bp = pred_b[i].reshape(1, P)
        p = _head_level(xp, wa, ba, w1, b1, w2, b2, wp, bp, H, W, W_o, W_im)
        t = p[:, :, :W, :].transpose(0, 3, 1, 2).reshape(B, 3, 15, H, W)
        outs.append((t[:, :, 0:4], t[:, :, 4:5], t[:, :, 5:10],
                     t[:, :, 10:14], t[:, :, 14:15]))
    return tuple(outs)


# speedup vs baseline: 1.4327x; 1.4327x over previous
import functools

import jax
import jax.numpy as jnp
from jax.experimental import pallas as pl
from jax.experimental.pallas import tpu as pltpu


def _head_body(xp_ref, wa_ref, ba_ref, w1_ref, s1_ref, b1_ref, w2_ref, s2_ref,
               b2_ref, wp_ref, bp_ref, o_ref, h0, h1, *, H, W, W_o, W_im):
    C = wa_ref.shape[1]
    F = wa_ref.shape[2]
    P = wp_ref.shape[2]
    M = H * W_o

    col = jax.lax.broadcasted_iota(jnp.int32, (M, F), 0) % W_o
    keep = col < W

    @pl.when(pl.program_id(0) == 0)
    def _():
        h0[...] = jnp.zeros((H + 2, W_im, C), jnp.bfloat16)
        h1[...] = jnp.zeros((H + 2, W_im, C), jnp.bfloat16)

    x = xp_ref[0, 1:H + 1, 1:W_o + 1, :].reshape(M, C)
    a = jnp.dot(x, wa_ref[0], preferred_element_type=jnp.float32)
    a = a + ba_ref[0]
    a = jnp.where(keep, a, 0.0)
    h0[1:H + 1, 1:W_o + 1, :] = a.astype(jnp.bfloat16).reshape(H, W_o, C)

    def conv3(src_ref, w_ref, s_ref, b_ref):
        acc = None
        for t in range(9):
            di, dj = t // 3, t % 3
            xs = src_ref[di:di + H, dj:dj + W_o, :].reshape(M, C)
            d = jnp.dot(xs, w_ref[0, t], preferred_element_type=jnp.float32)
            acc = d if acc is None else acc + d
        return jnp.maximum(acc * s_ref[0] + b_ref[0], 0.0)

    v1 = jnp.where(keep, conv3(h0, w1_ref, s1_ref, b1_ref), 0.0)
    h1[1:H + 1, 1:W_o + 1, :] = v1.astype(jnp.bfloat16).reshape(H, W_o, C)
    v2 = conv3(h1, w2_ref, s2_ref, b2_ref)

    p = jnp.dot(v2, wp_ref[0], preferred_element_type=jnp.float32)
    p = p + bp_ref[0]
    k = jax.lax.broadcasted_iota(jnp.int32, (M, P), 1) % 15
    p = jnp.where(k == 4, jax.nn.sigmoid(p),
                  jnp.where(k == 14, jax.nn.softplus(p) + 1.0, p))
    o_ref[0] = p.reshape(H, W_o, P)


def _head_level(i, xp, wa, ba, w1, s1, b1, w2, s2, b2, wp, bp, H, W, W_o,
                W_im):
    B = xp.shape[0]
    C = wa.shape[1]
    F = wa.shape[2]
    P = wp.shape[2]
    L = wa.shape[0]
    body = functools.partial(_head_body, H=H, W=W, W_o=W_o, W_im=W_im)
    return pl.pallas_call(
        body,
        grid=(B,),
        in_specs=[
            pl.BlockSpec((1, H + 2, W_im, C), lambda b: (b, 0, 0, 0)),
            pl.BlockSpec((1, C, F), lambda b, i=i: (i, 0, 0)),
            pl.BlockSpec((1, 1, F), lambda b, i=i: (i, 0, 0)),
            pl.BlockSpec((1, 9, C, F), lambda b, i=i: (i, 0, 0, 0)),
            pl.BlockSpec((1, 1, F), lambda b, i=i: (i, 0, 0)),
            pl.BlockSpec((1, 1, F), lambda b, i=i: (i, 0, 0)),
            pl.BlockSpec((1, 9, F, F), lambda b, i=i: (i, 0, 0, 0)),
            pl.BlockSpec((1, 1, F), lambda b, i=i: (i, 0, 0)),
            pl.BlockSpec((1, 1, F), lambda b, i=i: (i, 0, 0)),
            pl.BlockSpec((1, F, P), lambda b, i=i: (i, 0, 0)),
            pl.BlockSpec((1, 1, P), lambda b, i=i: (i, 0, 0)),
        ],
        out_specs=pl.BlockSpec((1, H, W_o, P), lambda b: (b, 0, 0, 0)),
        out_shape=jax.ShapeDtypeStruct((B, H, W_o, P), jnp.float32),
        scratch_shapes=[
            pltpu.VMEM((H + 2, W_im, C), jnp.bfloat16),
            pltpu.VMEM((H + 2, W_im, C), jnp.bfloat16),
        ],
        compiler_params=pltpu.CompilerParams(
            dimension_semantics=("arbitrary",)),
    )(xp, wa, ba, w1, s1, b1, w2, s2, b2, wp, bp)


def kernel(feat0, feat1, feat2, feat3, adapt_w, adapt_b, c1_w, c1_b, bn1_g,
           bn1_b, bn1_m, bn1_v, c2_w, c2_b, bn2_g, bn2_b, bn2_m, bn2_v,
           pred_w, pred_b):
    eps = 1e-5
    L, F = adapt_b.shape
    C = feat0.shape[1]
    P = pred_b.shape[1]
    feats = [feat0, feat1, feat2, feat3]
    bf16 = jnp.bfloat16

    wa = adapt_w.reshape(L, F, C).transpose(0, 2, 1).astype(bf16)
    w1 = c1_w.transpose(0, 3, 4, 2, 1).reshape(L, 9, C, F).astype(bf16)
    w2 = c2_w.transpose(0, 3, 4, 2, 1).reshape(L, 9, F, F).astype(bf16)
    wp = pred_w.reshape(L, P, F).transpose(0, 2, 1)

    s1 = (bn1_g / jnp.sqrt(bn1_v + eps)).reshape(L, 1, F)
    b1 = ((c1_b - bn1_m) * s1[:, 0] + bn1_b).reshape(L, 1, F)
    s2 = (bn2_g / jnp.sqrt(bn2_v + eps)).reshape(L, 1, F)
    b2 = ((c2_b - bn2_m) * s2[:, 0] + bn2_b).reshape(L, 1, F)
    ba = adapt_b.reshape(L, 1, F)
    bp = pred_b.reshape(L, 1, P)

    outs = []
    for i, f in enumerate(feats):
        B, _, H, W = f.shape
        W_o = -(-W // 16) * 16
        W_im = -(-(W_o + 2) // 16) * 16
        x = f.transpose(0, 2, 3, 1)
        xp = jnp.pad(x, ((0, 0), (1, 1), (1, W_im - W - 1), (0, 0)))
        xp = xp.astype(bf16)
        p = _head_level(i, xp, wa, ba, w1, s1, b1, w2, s2, b2, wp, bp,
                        H, W, W_o, W_im)
        t = p[:, :, :W, :].transpose(0, 3, 1, 2).reshape(B, 3, 15, H, W)
        outs.append((t[:, :, 0:4], t[:, :, 4:5], t[:, :, 5:10],
                     t[:, :, 10:14], t[:, :, 14:15]))
    return tuple(outs)


# speedup vs baseline: 1.5511x; 1.0826x over previous
import jax
import jax.numpy as jnp
from jax.experimental import pallas as pl
from jax.experimental.pallas import tpu as pltpu


def _geom(W):
    W_o = -(-W // 16) * 16
    W_im = -(-(W_o + 2) // 16) * 16
    return W_o, W_im


def _head_body(shapes, *refs):
    n = len(shapes)
    xp_refs = refs[:n]
    (wa_ref, ba_ref, w1_ref, s1_ref, b1_ref, w2_ref, s2_ref, b2_ref,
     wp_ref, bp_ref) = refs[n:n + 10]
    o_refs = refs[n + 10:n + 10 + n]
    h_refs = refs[n + 10 + n:]

    C = wa_ref.shape[1]
    F = wa_ref.shape[2]
    P = wp_ref.shape[2]

    @pl.when(pl.program_id(0) == 0)
    def _():
        for h in h_refs:
            h[...] = jnp.zeros(h.shape, jnp.bfloat16)

    for i, (H, W, W_o, W_im) in enumerate(shapes):
        xp_ref = xp_refs[i]
        o_ref = o_refs[i]
        h0 = h_refs[2 * i]
        h1 = h_refs[2 * i + 1]
        M = H * W_o

        col = jax.lax.broadcasted_iota(jnp.int32, (M, F), 0) % W_o
        keep = col < W

        x = xp_ref[0, 1:H + 1, 1:W_o + 1, :].reshape(M, C)
        a = jnp.dot(x, wa_ref[i], preferred_element_type=jnp.float32)
        a = a + ba_ref[i]
        a = jnp.where(keep, a, 0.0)
        h0[1:H + 1, 1:W_o + 1, :] = a.astype(jnp.bfloat16).reshape(H, W_o, C)

        def conv3(src_ref, w_ref, s_ref, b_ref):
            acc = None
            for t in range(9):
                di, dj = t // 3, t % 3
                xs = src_ref[di:di + H, dj:dj + W_o, :].reshape(M, C)
                d = jnp.dot(xs, w_ref[i, t],
                            preferred_element_type=jnp.float32)
                acc = d if acc is None else acc + d
            return jnp.maximum(acc * s_ref[i] + b_ref[i], 0.0)

        v1 = jnp.where(keep, conv3(h0, w1_ref, s1_ref, b1_ref), 0.0)
        h1[1:H + 1, 1:W_o + 1, :] = v1.astype(jnp.bfloat16).reshape(
            H, W_o, C)
        v2 = conv3(h1, w2_ref, s2_ref, b2_ref)

        p = jnp.dot(v2, wp_ref[i], preferred_element_type=jnp.float32)
        p = p + bp_ref[i]
        k = jax.lax.broadcasted_iota(jnp.int32, (M, P), 1) % 15
        p = jnp.where(k == 4, jax.nn.sigmoid(p),
                      jnp.where(k == 14, jax.nn.softplus(p) + 1.0, p))
        o_ref[0] = p.reshape(H, W_o, P)


def kernel(feat0, feat1, feat2, feat3, adapt_w, adapt_b, c1_w, c1_b, bn1_g,
           bn1_b, bn1_m, bn1_v, c2_w, c2_b, bn2_g, bn2_b, bn2_m, bn2_v,
           pred_w, pred_b):
    eps = 1e-5
    L, F = adapt_b.shape
    C = feat0.shape[1]
    P = pred_b.shape[1]
    feats = [feat0, feat1, feat2, feat3]
    bf16 = jnp.bfloat16
    B = feat0.shape[0]

    wa = adapt_w.reshape(L, F, C).transpose(0, 2, 1).astype(bf16)
    w1 = c1_w.transpose(0, 3, 4, 2, 1).reshape(L, 9, C, F).astype(bf16)
    w2 = c2_w.transpose(0, 3, 4, 2, 1).reshape(L, 9, F, F).astype(bf16)
    wp = pred_w.reshape(L, P, F).transpose(0, 2, 1)

    s1 = (bn1_g / jnp.sqrt(bn1_v + eps)).reshape(L, 1, F)
    b1 = ((c1_b - bn1_m) * s1[:, 0] + bn1_b).reshape(L, 1, F)
    s2 = (bn2_g / jnp.sqrt(bn2_v + eps)).reshape(L, 1, F)
    b2 = ((c2_b - bn2_m) * s2[:, 0] + bn2_b).reshape(L, 1, F)
    ba = adapt_b.reshape(L, 1, F)
    bp = pred_b.reshape(L, 1, P)

    shapes = []
    xps = []
    for f in feats:
        _, _, H, W = f.shape
        W_o, W_im = _geom(W)
        shapes.append((H, W, W_o, W_im))
        x = f.transpose(0, 2, 3, 1)
        xp = jnp.pad(x, ((0, 0), (1, 1), (1, W_im - W - 1), (0, 0)))
        xps.append(xp.astype(bf16))

    full3 = lambda a: pl.BlockSpec(a.shape, lambda b: (0, 0, 0))
    full4 = lambda a: pl.BlockSpec(a.shape, lambda b: (0, 0, 0, 0))
    in_specs = (
        [pl.BlockSpec((1, H + 2, W_im, C), lambda b: (b, 0, 0, 0))
         for (H, W, W_o, W_im) in shapes]
        + [full3(wa), full3(ba), full4(w1), full3(s1), full3(b1), full4(w2),
           full3(s2), full3(b2), full3(wp), full3(bp)]
    )
    out_specs = [pl.BlockSpec((1, H, W_o, P), lambda b: (b, 0, 0, 0))
                 for (H, W, W_o, W_im) in shapes]
    out_shape = [jax.ShapeDtypeStruct((B, H, W_o, P), jnp.float32)
                 for (H, W, W_o, W_im) in shapes]
    scratch_shapes = []
    for (H, W, W_o, W_im) in shapes:
        scratch_shapes += [pltpu.VMEM((H + 2, W_im, C), bf16)] * 2

    ps = pl.pallas_call(
        lambda *refs: _head_body(shapes, *refs),
        grid=(B,),
        in_specs=in_specs,
        out_specs=out_specs,
        out_shape=out_shape,
        scratch_shapes=scratch_shapes,
        compiler_params=pltpu.CompilerParams(
            dimension_semantics=("arbitrary",)),
    )(*xps, wa, ba, w1, s1, b1, w2, s2, b2, wp, bp)

    outs = []
    for (H, W, W_o, W_im), p in zip(shapes, ps):
        t = p[:, :, :W, :].transpose(0, 3, 1, 2).reshape(B, 3, 15, H, W)
        outs.append((t[:, :, 0:4], t[:, :, 4:5], t[:, :, 5:10],
                     t[:, :, 10:14], t[:, :, 14:15]))
    return tuple(outs)


# speedup vs baseline: 1.6393x; 1.0569x over previous
import jax
import jax.numpy as jnp
from jax.experimental import pallas as pl
from jax.experimental.pallas import tpu as pltpu


def _geom(W):
    W_o = -(-W // 16) * 16
    W_im = -(-(W_o + 2) // 16) * 16
    return W_o, W_im


def _head_body(shapes, *refs):
    n = len(shapes)
    x_refs = refs[:n]
    (wa_ref, ba_ref, w1_ref, s1_ref, b1_ref, w2_ref, s2_ref, b2_ref,
     wp_ref, bp_ref) = refs[n:n + 10]
    o_refs = refs[n + 10:n + 10 + 5 * n]
    h_refs = refs[n + 10 + 5 * n:]

    C = wa_ref.shape[1]
    F = wa_ref.shape[2]
    P = wp_ref.shape[1]

    @pl.when(pl.program_id(0) == 0)
    def _():
        for h in h_refs:
            h[...] = jnp.zeros(h.shape, jnp.bfloat16)

    for i, (H, W, W_o, W_im) in enumerate(shapes):
        x_ref = x_refs[i]
        ob, oo, os_, oa, od = o_refs[5 * i:5 * i + 5]
        h0 = h_refs[2 * i]
        h1 = h_refs[2 * i + 1]
        M = H * W_o

        if W_o != W:
            col = jax.lax.broadcasted_iota(jnp.int32, (M, F), 0) % W_o
            keep = col < W

        def to_im2col(v, dst):
            if W_o != W:
                v = jnp.where(keep, v, 0.0)
            img = v.astype(jnp.bfloat16).reshape(H, W_o, C)
            dst[1:H + 1, 1:W_o + 1, 0:C] = img
            dst[1:H + 1, 0:W_o, C:2 * C] = img
            dst[1:H + 1, 0:W_o - 1, 2 * C:3 * C] = img[:, 1:, :]

        x = x_ref[0].reshape(M, C)
        a = jnp.dot(x, wa_ref[i], preferred_element_type=jnp.float32)
        to_im2col(a + ba_ref[i], h0)

        def conv3(src_ref, w_ref, s_ref, b_ref):
            acc = None
            for di in range(3):
                xs = src_ref[di:di + H, 0:W_o, :].reshape(M, 3 * C)
                d = jnp.dot(xs, w_ref[i, di],
                            preferred_element_type=jnp.float32)
                acc = d if acc is None else acc + d
            return jnp.maximum(acc * s_ref[i] + b_ref[i], 0.0)

        to_im2col(conv3(h0, w1_ref, s1_ref, b1_ref), h1)
        v2 = conv3(h1, w2_ref, s2_ref, b2_ref)

        pT = jax.lax.dot_general(wp_ref[i], v2, (((1,), (1,)), ((), ())),
                                 preferred_element_type=jnp.float32)
        pT = pT + bp_ref[i]
        k = jax.lax.broadcasted_iota(jnp.int32, (P, M), 0) % 15
        pT = jnp.where(k == 4, jax.nn.sigmoid(pT),
                       jnp.where(k == 14, jax.nn.softplus(pT) + 1.0, pT))

        for aidx in range(3):
            base = 15 * aidx
            ob[0, 4 * aidx:4 * aidx + 4] = pT[base:base + 4]
            oo[0, aidx:aidx + 1] = pT[base + 4:base + 5]
            os_[0, 5 * aidx:5 * aidx + 5] = pT[base + 5:base + 10]
            oa[0, 4 * aidx:4 * aidx + 4] = pT[base + 10:base + 14]
            od[0, aidx:aidx + 1] = pT[base + 14:base + 15]


def kernel(feat0, feat1, feat2, feat3, adapt_w, adapt_b, c1_w, c1_b, bn1_g,
           bn1_b, bn1_m, bn1_v, c2_w, c2_b, bn2_g, bn2_b, bn2_m, bn2_v,
           pred_w, pred_b):
    eps = 1e-5
    L, F = adapt_b.shape
    C = feat0.shape[1]
    P = pred_b.shape[1]
    feats = [feat0, feat1, feat2, feat3]
    bf16 = jnp.bfloat16
    B = feat0.shape[0]

    wa = adapt_w.reshape(L, F, C).transpose(0, 2, 1).astype(bf16)
    w1 = c1_w.transpose(0, 3, 4, 2, 1).reshape(L, 3, 3 * C, F).astype(bf16)
    w2 = c2_w.transpose(0, 3, 4, 2, 1).reshape(L, 3, 3 * F, F).astype(bf16)
    wp = pred_w.reshape(L, P, F)

    s1 = (bn1_g / jnp.sqrt(bn1_v + eps)).reshape(L, 1, F)
    b1 = ((c1_b - bn1_m) * s1[:, 0] + bn1_b).reshape(L, 1, F)
    s2 = (bn2_g / jnp.sqrt(bn2_v + eps)).reshape(L, 1, F)
    b2 = ((c2_b - bn2_m) * s2[:, 0] + bn2_b).reshape(L, 1, F)
    ba = adapt_b.reshape(L, 1, F)
    bp = pred_b.reshape(L, P, 1)

    shapes = []
    xs = []
    for f in feats:
        _, _, H, W = f.shape
        W_o, W_im = _geom(W)
        shapes.append((H, W, W_o, W_im))
        x = f.transpose(0, 2, 3, 1)
        if W_o != W:
            x = jnp.pad(x, ((0, 0), (0, 0), (0, W_o - W), (0, 0)))
        xs.append(x.astype(bf16))

    full = lambda a: pl.BlockSpec(a.shape, lambda b: (0,) * a.ndim)
    in_specs = (
        [pl.BlockSpec((1, H, W_o, C), lambda b: (b, 0, 0, 0))
         for (H, W, W_o, W_im) in shapes]
        + [full(a) for a in (wa, ba, w1, s1, b1, w2, s2, b2, wp, bp)]
    )
    out_specs = []
    out_shape = []
    for (H, W, W_o, W_im) in shapes:
        for ch in (12, 3, 15, 12, 3):
            out_specs.append(pl.BlockSpec((1, ch, H * W_o),
                                          lambda b: (b, 0, 0)))
            out_shape.append(
                jax.ShapeDtypeStruct((B, ch, H * W_o), jnp.float32))
    scratch_shapes = []
    for (H, W, W_o, W_im) in shapes:
        scratch_shapes += [pltpu.VMEM((H + 2, W_im, 3 * C), bf16)] * 2

    ps = pl.pallas_call(
        lambda *refs: _head_body(shapes, *refs),
        grid=(B,),
        in_specs=in_specs,
        out_specs=out_specs,
        out_shape=out_shape,
        scratch_shapes=scratch_shapes,
        compiler_params=pltpu.CompilerParams(
            dimension_semantics=("arbitrary",)),
    )(*xs, wa, ba, w1, s1, b1, w2, s2, b2, wp, bp)

    outs = []
    for li, (H, W, W_o, W_im) in enumerate(shapes):
        leaves = []
        for j, ch in enumerate((12, 3, 15, 12, 3)):
            t = ps[5 * li + j].reshape(B, 3, ch // 3, H, W_o)
            if W_o != W:
                t = t[..., :W]
            leaves.append(t)
        outs.append(tuple(leaves))
    return tuple(outs)
